# baseline (device time: 23505 ns/iter reference)
import jax
import jax.numpy as jnp
from jax import lax
from jax.experimental import pallas as pl
from jax.experimental.pallas import tpu as pltpu

N_DEV = 4
N_LAYERS = 3
SEND_ORDER = [2, 1, 3]


def kernel(x, Win0, Wout0, Win1, Wout1, Win2, Wout2):
    b, d = x.shape
    _, hdim = Win0.shape
    rows = b // 2

    def body(
        x_ref,
        win0_ref,
        wout0_ref,
        win1_ref,
        wout1_ref,
        win2_ref,
        wout2_ref,
        out_ref,
        send_buf,
        comm_ref,
        win_bf,
        wout_bf,
        send_sems,
        recv_sems,
    ):
        my = lax.axis_index("i")
        wins = [win0_ref, win1_ref, win2_ref]
        wouts = [wout0_ref, wout1_ref, wout2_ref]
        all_rdmas = []

        barrier_sem = pltpu.get_barrier_semaphore()
        for off in range(1, N_DEV):
            pl.semaphore_signal(
                barrier_sem, inc=1,
                device_id=(lax.rem(my + off, N_DEV),),
                device_id_type=pl.DeviceIdType.MESH,
            )

        def compute_half(l, x_half_bf):
            h = jnp.dot(
                x_half_bf, win_bf[l, :, :], preferred_element_type=jnp.float32
            )
            h = jnp.maximum(h, 0.0).astype(jnp.bfloat16)
            return jnp.dot(
                h, wout_bf[l, :, :], preferred_element_type=jnp.float32
            )

        def start_sends(l, hf, partial):
            send_buf[l, hf, :, :] = partial.astype(jnp.bfloat16)
            rdmas = []
            for off in SEND_ORDER:
                peer = lax.rem(my + off, N_DEV)
                rdma = pltpu.make_async_remote_copy(
                    src_ref=send_buf.at[l, hf],
                    dst_ref=comm_ref.at[l, hf, N_DEV - 1 - off],
                    send_sem=send_sems.at[l, hf, off - 1],
                    recv_sem=recv_sems.at[l, hf, N_DEV - 1 - off],
                    device_id=(peer,),
                    device_id_type=pl.DeviceIdType.MESH,
                )
                rdma.start()
                rdmas.append(rdma)
            all_rdmas.extend(rdmas)
            return rdmas

        def wait_and_sum(l, hf, own_partial, rdmas):
            total = own_partial
            for i, off in enumerate(SEND_ORDER):
                rdmas[i].wait_recv()
                total = total + comm_ref[
                    l, hf, N_DEV - 1 - off, :, :
                ].astype(jnp.float32)
            return total

        win_bf[0, :, :] = wins[0][:, :].astype(jnp.bfloat16)
        wout_bf[0, :, :] = wouts[0][:, :].astype(jnp.bfloat16)
        x_bf = x_ref[:, :].astype(jnp.bfloat16)

        own_top = compute_half(0, x_bf[:rows, :])
        pl.semaphore_wait(barrier_sem, N_DEV - 1)
        rd_top = start_sends(0, 0, own_top)
        own_bot = compute_half(0, x_bf[rows:, :])
        rd_bot = start_sends(0, 1, own_bot)

        for l in range(N_LAYERS):
            if l + 1 < N_LAYERS:
                win_bf[l + 1, :, :] = wins[l + 1][:, :].astype(jnp.bfloat16)
                wout_bf[l + 1, :, :] = wouts[l + 1][:, :].astype(jnp.bfloat16)

            tot_top = wait_and_sum(l, 0, own_top, rd_top)
            if l + 1 < N_LAYERS:
                own_top = compute_half(l + 1, tot_top.astype(jnp.bfloat16))
                next_rd_top = start_sends(l + 1, 0, own_top)
            else:
                out_ref[:rows, :] = tot_top

            tot_bot = wait_and_sum(l, 1, own_bot, rd_bot)
            if l + 1 < N_LAYERS:
                own_bot = compute_half(l + 1, tot_bot.astype(jnp.bfloat16))
                rd_bot = start_sends(l + 1, 1, own_bot)
                rd_top = next_rd_top
            else:
                out_ref[rows:, :] = tot_bot

        for r in all_rdmas:
            r.wait_send()

    return pl.pallas_call(
        body,
        out_shape=jax.ShapeDtypeStruct((b, d), jnp.float32),
        in_specs=[pl.BlockSpec(memory_space=pltpu.VMEM)] * 7,
        out_specs=pl.BlockSpec(memory_space=pltpu.VMEM),
        scratch_shapes=[
            pltpu.VMEM((N_LAYERS, 2, rows, d), jnp.bfloat16),
            pltpu.VMEM((N_LAYERS, 2, N_DEV - 1, rows, d), jnp.bfloat16),
            pltpu.VMEM((N_LAYERS, d, hdim), jnp.bfloat16),
            pltpu.VMEM((N_LAYERS, hdim, d), jnp.bfloat16),
            pltpu.SemaphoreType.DMA((N_LAYERS, 2, N_DEV - 1)),
            pltpu.SemaphoreType.DMA((N_LAYERS, 2, N_DEV - 1)),
        ],
        compiler_params=pltpu.CompilerParams(collective_id=0),
    )(x, Win0, Wout0, Win1, Wout1, Win2, Wout2)


# device time: 23057 ns/iter; 1.0194x vs baseline; 1.0194x over previous
import jax
import jax.numpy as jnp
from jax import lax
from jax.experimental import pallas as pl
from jax.experimental.pallas import tpu as pltpu

N_DEV = 4
N_LAYERS = 3
NSPLIT = 4
SEND_ORDER = [2, 1, 3]


def kernel(x, Win0, Wout0, Win1, Wout1, Win2, Wout2):
    b, d = x.shape
    _, hdim = Win0.shape
    rows = b // NSPLIT

    def body(
        x_ref,
        win0_ref,
        wout0_ref,
        win1_ref,
        wout1_ref,
        win2_ref,
        wout2_ref,
        out_ref,
        send_buf,
        comm_ref,
        win_bf,
        wout_bf,
        send_sems,
        recv_sems,
    ):
        my = lax.axis_index("i")
        wins = [win0_ref, win1_ref, win2_ref]
        wouts = [wout0_ref, wout1_ref, wout2_ref]
        all_rdmas = []

        barrier_sem = pltpu.get_barrier_semaphore()
        for off in range(1, N_DEV):
            pl.semaphore_signal(
                barrier_sem, inc=1,
                device_id=(lax.rem(my + off, N_DEV),),
                device_id_type=pl.DeviceIdType.MESH,
            )

        def compute_chunk(l, x_chunk_bf):
            h = jnp.dot(
                x_chunk_bf, win_bf[l, :, :], preferred_element_type=jnp.float32
            )
            h = jnp.maximum(h, 0.0).astype(jnp.bfloat16)
            return jnp.dot(
                h, wout_bf[l, :, :], preferred_element_type=jnp.float32
            )

        def start_sends(l, c, partial):
            send_buf[l, c, :, :] = partial.astype(jnp.bfloat16)
            rdmas = []
            for off in SEND_ORDER:
                peer = lax.rem(my + off, N_DEV)
                rdma = pltpu.make_async_remote_copy(
                    src_ref=send_buf.at[l, c],
                    dst_ref=comm_ref.at[l, c, N_DEV - 1 - off],
                    send_sem=send_sems.at[l, c, off - 1],
                    recv_sem=recv_sems.at[l, c, N_DEV - 1 - off],
                    device_id=(peer,),
                    device_id_type=pl.DeviceIdType.MESH,
                )
                rdma.start()
                rdmas.append(rdma)
            all_rdmas.extend(rdmas)
            return rdmas

        def wait_and_sum(l, c, own_partial, rdmas):
            total = own_partial
            for i, off in enumerate(SEND_ORDER):
                rdmas[i].wait_recv()
                total = total + comm_ref[
                    l, c, N_DEV - 1 - off, :, :
                ].astype(jnp.float32)
            return total

        win_bf[0, :, :] = wins[0][:, :].astype(jnp.bfloat16)
        wout_bf[0, :, :] = wouts[0][:, :].astype(jnp.bfloat16)
        x_bf = x_ref[:, :].astype(jnp.bfloat16)

        own = [None] * NSPLIT
        rd = [None] * NSPLIT
        for c in range(NSPLIT):
            own[c] = compute_chunk(0, x_bf[c * rows:(c + 1) * rows, :])
            if c == 0:
                pl.semaphore_wait(barrier_sem, N_DEV - 1)
            rd[c] = start_sends(0, c, own[c])

        for l in range(N_LAYERS):
            if l + 1 < N_LAYERS:
                win_bf[l + 1, :, :] = wins[l + 1][:, :].astype(jnp.bfloat16)
                wout_bf[l + 1, :, :] = wouts[l + 1][:, :].astype(jnp.bfloat16)

            next_rd = [None] * NSPLIT
            for c in range(NSPLIT):
                tot = wait_and_sum(l, c, own[c], rd[c])
                if l + 1 < N_LAYERS:
                    own[c] = compute_chunk(l + 1, tot.astype(jnp.bfloat16))
                    next_rd[c] = start_sends(l + 1, c, own[c])
                else:
                    out_ref[c * rows:(c + 1) * rows, :] = tot
            rd = next_rd

        for r in all_rdmas:
            r.wait_send()

    return pl.pallas_call(
        body,
        out_shape=jax.ShapeDtypeStruct((b, d), jnp.float32),
        in_specs=[pl.BlockSpec(memory_space=pltpu.VMEM)] * 7,
        out_specs=pl.BlockSpec(memory_space=pltpu.VMEM),
        scratch_shapes=[
            pltpu.VMEM((N_LAYERS, NSPLIT, rows, d), jnp.bfloat16),
            pltpu.VMEM((N_LAYERS, NSPLIT, N_DEV - 1, rows, d), jnp.bfloat16),
            pltpu.VMEM((N_LAYERS, d, hdim), jnp.bfloat16),
            pltpu.VMEM((N_LAYERS, hdim, d), jnp.bfloat16),
            pltpu.SemaphoreType.DMA((N_LAYERS, NSPLIT, N_DEV - 1)),
            pltpu.SemaphoreType.DMA((N_LAYERS, NSPLIT, N_DEV - 1)),
        ],
        compiler_params=pltpu.CompilerParams(collective_id=0),
    )(x, Win0, Wout0, Win1, Wout1, Win2, Wout2)
